# baseline (speedup 1.0000x reference)
import functools
import jax
import jax.numpy as jnp
from jax import lax
from jax.experimental import pallas as pl
from jax.experimental.pallas import tpu as pltpu
from jax.experimental.pallas import tpu_sc as plsc

K = 20
B = 50
N = B * K
NW = 32
F32 = jnp.float32


def _body(xp_hbm, ap_hbm, w1a_hbm, b1a_hbm, w1bt_hbm, b1b_hbm,
          w2a_hbm, b2a_hbm, w2bs_hbm, b2bs_hbm,
          wh1_hbm, bh1_hbm, wh2_hbm, bh2_hbm,
          out_hbm,
          xs, attrT, linT, h1T, aggrT,
          w1a, b1a, w1bt, b1b, w2a, b2a, w2bs, b2bs,
          wh1, bh1, wh2, bh2v, outv):
    wid = lax.axis_index("s") * 2 + lax.axis_index("c")

    pltpu.sync_copy(w1a_hbm, w1a)
    pltpu.sync_copy(b1a_hbm, b1a)
    pltpu.sync_copy(w1bt_hbm, w1bt)
    pltpu.sync_copy(b1b_hbm, b1b)
    pltpu.sync_copy(w2a_hbm, w2a)
    pltpu.sync_copy(b2a_hbm, b2a)
    pltpu.sync_copy(w2bs_hbm, w2bs)
    pltpu.sync_copy(b2bs_hbm, b2bs)
    pltpu.sync_copy(wh1_hbm, wh1)
    pltpu.sync_copy(bh1_hbm, bh1)
    pltpu.sync_copy(wh2_hbm, wh2)
    pltpu.sync_copy(bh2_hbm, bh2v)

    lanes = lax.iota(jnp.int32, 16)
    zeros = lanes.astype(F32) * 0.0

    def do_graph(g):
        pltpu.sync_copy(xp_hbm.at[g], xs)
        pltpu.sync_copy(ap_hbm.at[g], attrT)

        def layer(_l, _c):
            def lin_n(n, c):
                xrow = xs[n, :]
                lo = b1a[0:16]
                hi = b1a[16:32]
                for k in range(9):
                    s = xrow[k]
                    lo = lo + w1a[k, 0:16] * s
                    hi = hi + w1a[k, 16:32] * s
                linT[n, 0:16] = lo
                linT[n, 16:32] = hi
                return c
            lax.fori_loop(0, K, lin_n, 0)

            def clr(c, cc):
                aggrT[pl.ds(c * 16, 16)] = zeros
                return cc
            lax.fori_loop(0, 64, clr, 0)

            def msg_j(j, c):
                a0 = attrT[pl.ds(j * 32, 16)]
                a1 = attrT[pl.ds(j * 32 + 16, 16)]
                m0 = lanes != j
                m1 = jnp.logical_and(lanes + 16 < K, lanes + 16 != j)
                l0 = linT[j, 0:16]
                l1 = linT[j, 16:32]
                w9lo = w1a[9, 0:16]
                w9hi = w1a[9, 16:32]
                h0 = []
                for cc in range(32):
                    s1 = w9lo[cc] if cc < 16 else w9hi[cc - 16]
                    s2 = l0[cc] if cc < 16 else l1[cc - 16]
                    h0.append(jnp.maximum(a0 * s1 + s2, 0.0))
                    h1T[cc, :] = jnp.maximum(a1 * s1 + s2, 0.0)
                b1blo = b1b[0:16]
                b1bhi = b1b[16:32]
                for cp in range(32):
                    bs = b1blo[cp] if cp < 16 else b1bhi[cp - 16]
                    acc0 = jnp.full((16,), bs, F32)
                    acc1 = acc0
                    wlo = w1bt[cp, 0:16]
                    whi = w1bt[cp, 16:32]
                    for cc in range(32):
                        s = wlo[cc] if cc < 16 else whi[cc - 16]
                        acc0 = acc0 + h0[cc] * s
                        acc1 = acc1 + h1T[cc, :] * s
                    acc0 = jnp.where(m0, jnp.maximum(acc0, 0.0), 0.0)
                    acc1 = jnp.where(m1, jnp.maximum(acc1, 0.0), 0.0)
                    a_lo = pl.ds(cp * 32, 16)
                    a_hi = pl.ds(cp * 32 + 16, 16)
                    aggrT[a_lo] = jnp.maximum(aggrT[a_lo], acc0)
                    aggrT[a_hi] = jnp.maximum(aggrT[a_hi], acc1)
                return c
            lax.fori_loop(0, K, msg_j, 0)

            for n in range(K):
                xrow = xs[n, :]
                acc = b2a[:]
                for k in range(9):
                    acc = acc + w2a[k, :] * xrow[k]
                for k in range(32):
                    arow = (aggrT[pl.ds(k * 32, 16)] if n < 16
                            else aggrT[pl.ds(k * 32 + 16, 16)])
                    acc = acc + w2a[9 + k, :] * arow[n % 16]
                h2v = jnp.maximum(acc, 0.0)
                acc2 = b2bs[:]
                for k in range(16):
                    acc2 = acc2 + w2bs[k, :] * h2v[k]
                comb = jnp.maximum(acc2, 0.0)
                xs[n, :] = jnp.where(lanes == 0, xrow[0], comb)
            return _c
        lax.fori_loop(0, 3, layer, 0)

        def head_n(n, carry):
            o0, o1 = carry
            xrow = xs[n, :]
            acc = bh1[:]
            for m in range(8):
                acc = acc + wh1[m, :] * xrow[1 + m]
            hh = jnp.maximum(acc, 0.0)
            wv = wh2[:]
            z = bh2v[:][0]
            for m2 in range(16):
                z = z + hh[m2] * wv[m2]
            sig = 1.0 / (1.0 + jnp.exp(jnp.full((16,), -z, F32)))
            o0 = jnp.where(lanes == n, sig, o0)
            o1 = jnp.where(lanes + 16 == n, sig, o1)
            return (o0, o1)
        o0, o1 = lax.fori_loop(0, K, head_n, (zeros, zeros))
        outv[0:16] = o0
        outv[16:32] = o1
        pltpu.sync_copy(outv, out_hbm.at[g])

    def gloop(gs, c):
        g = wid + gs * NW

        @pl.when(g < B)
        def _():
            do_graph(g)
        return c
    lax.fori_loop(0, 2, gloop, 0)


def kernel(x, edge_index, edge_attr,
           W1a, b1a, W1b, b1b, W2a, b2a, W2b, b2b,
           Wh1, bh1, Wh2, bh2):
    xp = jnp.pad(x, ((0, 0), (0, 7))).reshape(B, K, 16)
    idx = []
    msk = []
    for p in range(K * 32):
        j, i = divmod(p, 32)
        if i < K and i != j:
            idx.append(i * (K - 1) + (j if j < i else j - 1))
            msk.append(True)
        else:
            idx.append(0)
            msk.append(False)
    ea = edge_attr.reshape(B, K * (K - 1))
    ap = jnp.where(jnp.asarray(msk)[None, :],
                   ea[:, jnp.asarray(idx, jnp.int32)], 0.0)
    w2bs = jnp.zeros((16, 16), F32).at[:, 1:9].set(W2b)
    b2bs = jnp.zeros((16,), F32).at[1:9].set(b2b)
    bh2v = jnp.full((16,), bh2[0], F32)

    mesh = plsc.VectorSubcoreMesh(core_axis_name="c", subcore_axis_name="s")
    run = functools.partial(
        pl.kernel, mesh=mesh,
        out_type=jax.ShapeDtypeStruct((B, 32), F32),
        scratch_types=[
            pltpu.VMEM((K, 16), F32),
            pltpu.VMEM((K * 32,), F32),
            pltpu.VMEM((K, 32), F32),
            pltpu.VMEM((32, 16), F32),
            pltpu.VMEM((32 * 32,), F32),
            pltpu.VMEM((10, 32), F32),
            pltpu.VMEM((32,), F32),
            pltpu.VMEM((32, 32), F32),
            pltpu.VMEM((32,), F32),
            pltpu.VMEM((41, 16), F32),
            pltpu.VMEM((16,), F32),
            pltpu.VMEM((16, 16), F32),
            pltpu.VMEM((16,), F32),
            pltpu.VMEM((8, 16), F32),
            pltpu.VMEM((16,), F32),
            pltpu.VMEM((16,), F32),
            pltpu.VMEM((16,), F32),
            pltpu.VMEM((32,), F32),
        ],
    )(_body)
    out = run(xp, ap, W1a, b1a, W1b.T, b1b, W2a, b2a, w2bs, b2bs,
              Wh1, bh1, Wh2.reshape(16), bh2v)
    return out[:, :K].reshape(N, 1)


# speedup vs baseline: 2.0783x; 2.0783x over previous
import functools
import jax
import jax.numpy as jnp
from jax import lax
from jax.experimental import pallas as pl
from jax.experimental.pallas import tpu as pltpu
from jax.experimental.pallas import tpu_sc as plsc

K = 20
B = 50
N = B * K
NW = 32
F32 = jnp.float32


def _body(xp_hbm, ap_hbm, w1a_hbm, b1a_hbm, w1bt_hbm, b1b_hbm,
          w2a_hbm, b2a_hbm, w2bs_hbm, b2bs_hbm,
          wh1_hbm, bh1_hbm, wh2_hbm, bh2_hbm,
          out_hbm,
          xs, attrT, linT, accQ, aggrQ, aggrT,
          w1a, b1a, w1bt, b1b, w2a, b2a, w2bs, b2bs,
          wh1, bh1, wh2, bh2v, outv):
    wid = lax.axis_index("s") * 2 + lax.axis_index("c")

    pltpu.sync_copy(w1a_hbm, w1a)
    pltpu.sync_copy(b1a_hbm, b1a)
    pltpu.sync_copy(w1bt_hbm, w1bt)
    pltpu.sync_copy(b1b_hbm, b1b)
    pltpu.sync_copy(w2a_hbm, w2a)
    pltpu.sync_copy(b2a_hbm, b2a)
    pltpu.sync_copy(w2bs_hbm, w2bs)
    pltpu.sync_copy(b2bs_hbm, b2bs)
    pltpu.sync_copy(wh1_hbm, wh1)
    pltpu.sync_copy(bh1_hbm, bh1)
    pltpu.sync_copy(wh2_hbm, wh2)
    pltpu.sync_copy(bh2_hbm, bh2v)

    lanes = lax.iota(jnp.int32, 16)
    zeros = lanes.astype(F32) * 0.0

    def do_graph(g):
        pltpu.sync_copy(xp_hbm.at[g], xs)
        pltpu.sync_copy(ap_hbm.at[g], attrT)

        def layer(_l, _c):
            def lin_n(n, c):
                xrow = xs[n, :]
                lo = b1a[0:16]
                hi = b1a[16:32]
                for k in range(9):
                    s = xrow[k]
                    lo = lo + w1a[k, 0:16] * s
                    hi = hi + w1a[k, 16:32] * s
                linT[n, 0:16] = lo
                linT[n, 16:32] = hi
                return c
            lax.fori_loop(0, K, lin_n, 0)

            def clr(c, cc):
                aggrT[pl.ds(c * 16, 16)] = zeros
                return cc
            lax.fori_loop(0, 64, clr, 0)

            def clrq(c, cc):
                aggrQ[pl.ds(c * 16, 16)] = zeros
                return cc
            lax.fori_loop(0, 32, clrq, 0)

            def msg_j(j, c):
                a0 = attrT[pl.ds(j * 32, 16)]
                m0 = lanes != j
                l0 = linT[j, 0:16]
                l1 = linT[j, 16:32]
                w9lo = w1a[9, 0:16]
                w9hi = w1a[9, 16:32]
                h0 = []
                for cc in range(32):
                    s1 = w9lo[cc] if cc < 16 else w9hi[cc - 16]
                    s2 = l0[cc] if cc < 16 else l1[cc - 16]
                    h0.append(jnp.maximum(a0 * s1 + s2, 0.0))
                b1blo = b1b[0:16]
                b1bhi = b1b[16:32]
                for cp in range(32):
                    bs = b1blo[cp] if cp < 16 else b1bhi[cp - 16]
                    acc0 = jnp.full((16,), bs, F32)
                    wlo = w1bt[cp, 0:16]
                    whi = w1bt[cp, 16:32]
                    for cc in range(32):
                        s = wlo[cc] if cc < 16 else whi[cc - 16]
                        acc0 = acc0 + h0[cc] * s
                    acc0 = jnp.where(m0, jnp.maximum(acc0, 0.0), 0.0)
                    a_lo = pl.ds(cp * 32, 16)
                    aggrT[a_lo] = jnp.maximum(aggrT[a_lo], acc0)
                return c
            lax.fori_loop(0, K, msg_j, 0)

            dq = jnp.bitwise_and(lanes, 3)
            sq = lax.shift_right_logical(lanes, 2)
            w9lo = w1a[9, 0:16]
            w9hi = w1a[9, 16:32]
            b1blo = b1b[0:16]
            b1bhi = b1b[16:32]
            def quad_jp(jp, c):
                aq = attrT[pl.ds(640 + jp * 16, 16)]
                mq = (16 + dq) != (4 * jp + sq)
                for half in range(2):
                    lrows = [linT[4 * jp + s, pl.ds(half * 16, 16)]
                             for s in range(4)]
                    w9h = w9lo if half == 0 else w9hi
                    hq = []
                    for ci in range(16):
                        ls = jnp.where(sq == 0, lrows[0][ci],
                             jnp.where(sq == 1, lrows[1][ci],
                             jnp.where(sq == 2, lrows[2][ci], lrows[3][ci])))
                        hq.append(jnp.maximum(aq * w9h[ci] + ls, 0.0))
                    for cp in range(32):
                        wh = w1bt[cp, pl.ds(half * 16, 16)]
                        qs = pl.ds(cp * 16, 16)
                        if half == 0:
                            bs = b1blo[cp] if cp < 16 else b1bhi[cp - 16]
                            acc = jnp.full((16,), bs, F32)
                        else:
                            acc = accQ[qs]
                        for ci in range(16):
                            acc = acc + hq[ci] * wh[ci]
                        if half == 0:
                            accQ[qs] = acc
                        else:
                            acc = jnp.where(mq, jnp.maximum(acc, 0.0), 0.0)
                            aggrQ[qs] = jnp.maximum(aggrQ[qs], acc)
                return c
            lax.fori_loop(0, 5, quad_jp, 0)

            def fold_cp(cp, c):
                q = aggrQ[pl.ds(cp * 16, 16)]
                r = zeros
                for d in range(4):
                    md = jnp.maximum(jnp.maximum(q[d], q[4 + d]),
                                     jnp.maximum(q[8 + d], q[12 + d]))
                    r = jnp.where(lanes == d, md, r)
                aggrT[pl.ds(cp * 32 + 16, 16)] = r
                return c
            lax.fori_loop(0, 32, fold_cp, 0)

            for n in range(K):
                xrow = xs[n, :]
                acc = b2a[:]
                for k in range(9):
                    acc = acc + w2a[k, :] * xrow[k]
                for k in range(32):
                    arow = (aggrT[pl.ds(k * 32, 16)] if n < 16
                            else aggrT[pl.ds(k * 32 + 16, 16)])
                    acc = acc + w2a[9 + k, :] * arow[n % 16]
                h2v = jnp.maximum(acc, 0.0)
                acc2 = b2bs[:]
                for k in range(16):
                    acc2 = acc2 + w2bs[k, :] * h2v[k]
                comb = jnp.maximum(acc2, 0.0)
                xs[n, :] = jnp.where(lanes == 0, xrow[0], comb)
            return _c
        lax.fori_loop(0, 3, layer, 0)

        def head_n(n, carry):
            o0, o1 = carry
            xrow = xs[n, :]
            acc = bh1[:]
            for m in range(8):
                acc = acc + wh1[m, :] * xrow[1 + m]
            hh = jnp.maximum(acc, 0.0)
            wv = wh2[:]
            z = bh2v[:][0]
            for m2 in range(16):
                z = z + hh[m2] * wv[m2]
            sig = 1.0 / (1.0 + jnp.exp(jnp.full((16,), -z, F32)))
            o0 = jnp.where(lanes == n, sig, o0)
            o1 = jnp.where(lanes + 16 == n, sig, o1)
            return (o0, o1)
        o0, o1 = lax.fori_loop(0, K, head_n, (zeros, zeros))
        outv[0:16] = o0
        outv[16:32] = o1
        pltpu.sync_copy(outv, out_hbm.at[g])

    def gloop(gs, c):
        g = wid + gs * NW

        @pl.when(g < B)
        def _():
            do_graph(g)
        return c
    lax.fori_loop(0, 2, gloop, 0)


def kernel(x, edge_index, edge_attr,
           W1a, b1a, W1b, b1b, W2a, b2a, W2b, b2b,
           Wh1, bh1, Wh2, bh2):
    xp = jnp.pad(x, ((0, 0), (0, 7))).reshape(B, K, 16)
    idx = []
    msk = []
    for p in range(K * 32):
        j, i = divmod(p, 32)
        if i < K and i != j:
            idx.append(i * (K - 1) + (j if j < i else j - 1))
            msk.append(True)
        else:
            idx.append(0)
            msk.append(False)
    for jp in range(5):
        for lane in range(16):
            s, d = divmod(lane, 4)
            j, i = 4 * jp + s, 16 + d
            if i != j:
                idx.append(i * (K - 1) + (j if j < i else j - 1))
                msk.append(True)
            else:
                idx.append(0)
                msk.append(False)
    ea = edge_attr.reshape(B, K * (K - 1))
    ap = jnp.where(jnp.asarray(msk)[None, :],
                   ea[:, jnp.asarray(idx, jnp.int32)], 0.0)
    w2bs = jnp.zeros((16, 16), F32).at[:, 1:9].set(W2b)
    b2bs = jnp.zeros((16,), F32).at[1:9].set(b2b)
    bh2v = jnp.full((16,), bh2[0], F32)

    mesh = plsc.VectorSubcoreMesh(core_axis_name="c", subcore_axis_name="s")
    run = functools.partial(
        pl.kernel, mesh=mesh,
        out_type=jax.ShapeDtypeStruct((B, 32), F32),
        scratch_types=[
            pltpu.VMEM((K, 16), F32),
            pltpu.VMEM((720,), F32),
            pltpu.VMEM((K, 32), F32),
            pltpu.VMEM((512,), F32),
            pltpu.VMEM((512,), F32),
            pltpu.VMEM((32 * 32,), F32),
            pltpu.VMEM((10, 32), F32),
            pltpu.VMEM((32,), F32),
            pltpu.VMEM((32, 32), F32),
            pltpu.VMEM((32,), F32),
            pltpu.VMEM((41, 16), F32),
            pltpu.VMEM((16,), F32),
            pltpu.VMEM((16, 16), F32),
            pltpu.VMEM((16,), F32),
            pltpu.VMEM((8, 16), F32),
            pltpu.VMEM((16,), F32),
            pltpu.VMEM((16,), F32),
            pltpu.VMEM((16,), F32),
            pltpu.VMEM((32,), F32),
        ],
    )(_body)
    out = run(xp, ap, W1a, b1a, W1b.T, b1b, W2a, b2a, w2bs, b2bs,
              Wh1, bh1, Wh2.reshape(16), bh2v)
    return out[:, :K].reshape(N, 1)


# speedup vs baseline: 2.0861x; 1.0038x over previous
import functools
import jax
import jax.numpy as jnp
from jax import lax
from jax.experimental import pallas as pl
from jax.experimental.pallas import tpu as pltpu
from jax.experimental.pallas import tpu_sc as plsc

K = 20
B = 50
N = B * K
NW = 32
F32 = jnp.float32


def _body(xp_hbm, ap_hbm, w1a_hbm, b1a_hbm, w1bt_hbm, b1b_hbm,
          w2a_hbm, b2a_hbm, w2bs_hbm, b2bs_hbm,
          wh1_hbm, bh1_hbm, wh2_hbm, bh2_hbm,
          out_hbm,
          xs, attrT, linT, accQ, aggrQ, aggrT,
          w1a, b1a, w1bt, b1b, w2a, b2a, w2bs, b2bs,
          wh1, bh1, wh2, bh2v, outv):
    wid = lax.axis_index("s") * 2 + lax.axis_index("c")

    pltpu.sync_copy(w1a_hbm, w1a)
    pltpu.sync_copy(b1a_hbm, b1a)
    pltpu.sync_copy(w1bt_hbm, w1bt)
    pltpu.sync_copy(b1b_hbm, b1b)
    pltpu.sync_copy(w2a_hbm, w2a)
    pltpu.sync_copy(b2a_hbm, b2a)
    pltpu.sync_copy(w2bs_hbm, w2bs)
    pltpu.sync_copy(b2bs_hbm, b2bs)
    pltpu.sync_copy(wh1_hbm, wh1)
    pltpu.sync_copy(bh1_hbm, bh1)
    pltpu.sync_copy(wh2_hbm, wh2)
    pltpu.sync_copy(bh2_hbm, bh2v)

    lanes = lax.iota(jnp.int32, 16)
    zeros = lanes.astype(F32) * 0.0
    dq = jnp.bitwise_and(lanes, 3)
    sq = lax.shift_right_logical(lanes, 2)
    w9lo = w1a[9, 0:16]
    w9hi = w1a[9, 16:32]
    b1blo = b1b[0:16]
    b1bhi = b1b[16:32]

    def do_graph(g):
        pltpu.sync_copy(xp_hbm.at[g], xs)
        pltpu.sync_copy(ap_hbm.at[g], attrT)

        def layer(_l, _c):
            def lin_n(n, c):
                xrow = xs[n, :]
                lo = b1a[0:16]
                hi = b1a[16:32]
                for k in range(9):
                    s = xrow[k]
                    lo = lo + w1a[k, 0:16] * s
                    hi = hi + w1a[k, 16:32] * s
                linT[n, 0:16] = lo
                linT[n, 16:32] = hi
                return c
            lax.fori_loop(0, K, lin_n, 0)

            def clr(c, cc):
                aggrT[pl.ds(c * 16, 16)] = zeros
                return cc
            lax.fori_loop(0, 64, clr, 0)

            def clrq(c, cc):
                aggrQ[pl.ds(c * 16, 16)] = zeros
                return cc
            lax.fori_loop(0, 32, clrq, 0)

            def msg_j(j, c):
                a0 = attrT[pl.ds(j * 32, 16)]
                m0 = lanes != j
                l0 = linT[j, 0:16]
                l1 = linT[j, 16:32]
                h0 = []
                for cc in range(32):
                    s1 = w9lo[cc] if cc < 16 else w9hi[cc - 16]
                    s2 = l0[cc] if cc < 16 else l1[cc - 16]
                    h0.append(jnp.maximum(a0 * s1 + s2, 0.0))
                for cp in range(32):
                    bs = b1blo[cp] if cp < 16 else b1bhi[cp - 16]
                    acc0 = jnp.full((16,), bs, F32)
                    wlo = w1bt[cp, 0:16]
                    whi = w1bt[cp, 16:32]
                    for cc in range(32):
                        s = wlo[cc] if cc < 16 else whi[cc - 16]
                        acc0 = acc0 + h0[cc] * s
                    acc0 = jnp.where(m0, jnp.maximum(acc0, 0.0), 0.0)
                    a_lo = pl.ds(cp * 32, 16)
                    aggrT[a_lo] = jnp.maximum(aggrT[a_lo], acc0)
                return c
            lax.fori_loop(0, K, msg_j, 0)

            def quad_jp(jp, c):
                aq = attrT[pl.ds(640 + jp * 16, 16)]
                mq = (16 + dq) != (4 * jp + sq)
                for half in range(2):
                    lrows = [linT[4 * jp + s, pl.ds(half * 16, 16)]
                             for s in range(4)]
                    w9h = w9lo if half == 0 else w9hi
                    hq = []
                    for ci in range(16):
                        ls = jnp.where(sq == 0, lrows[0][ci],
                             jnp.where(sq == 1, lrows[1][ci],
                             jnp.where(sq == 2, lrows[2][ci], lrows[3][ci])))
                        hq.append(jnp.maximum(aq * w9h[ci] + ls, 0.0))
                    for cp in range(32):
                        wh = w1bt[cp, pl.ds(half * 16, 16)]
                        qs = pl.ds(cp * 16, 16)
                        if half == 0:
                            bs = b1blo[cp] if cp < 16 else b1bhi[cp - 16]
                            acc = jnp.full((16,), bs, F32)
                        else:
                            acc = accQ[qs]
                        for ci in range(16):
                            acc = acc + hq[ci] * wh[ci]
                        if half == 0:
                            accQ[qs] = acc
                        else:
                            acc = jnp.where(mq, jnp.maximum(acc, 0.0), 0.0)
                            aggrQ[qs] = jnp.maximum(aggrQ[qs], acc)
                return c
            lax.fori_loop(0, 5, quad_jp, 0)

            def fold_cp(cp, c):
                q = aggrQ[pl.ds(cp * 16, 16)]
                r = zeros
                for d in range(4):
                    md = jnp.maximum(jnp.maximum(q[d], q[4 + d]),
                                     jnp.maximum(q[8 + d], q[12 + d]))
                    r = jnp.where(lanes == d, md, r)
                aggrT[pl.ds(cp * 32 + 16, 16)] = r
                return c
            lax.fori_loop(0, 32, fold_cp, 0)

            for n in range(K):
                xrow = xs[n, :]
                acc = b2a[:]
                for k in range(9):
                    acc = acc + w2a[k, :] * xrow[k]
                for k in range(32):
                    arow = (aggrT[pl.ds(k * 32, 16)] if n < 16
                            else aggrT[pl.ds(k * 32 + 16, 16)])
                    acc = acc + w2a[9 + k, :] * arow[n % 16]
                h2v = jnp.maximum(acc, 0.0)
                acc2 = b2bs[:]
                for k in range(16):
                    acc2 = acc2 + w2bs[k, :] * h2v[k]
                comb = jnp.maximum(acc2, 0.0)
                xs[n, :] = jnp.where(lanes == 0, xrow[0], comb)
            return _c
        lax.fori_loop(0, 3, layer, 0)

        def head_n(n, carry):
            o0, o1 = carry
            xrow = xs[n, :]
            acc = bh1[:]
            for m in range(8):
                acc = acc + wh1[m, :] * xrow[1 + m]
            hh = jnp.maximum(acc, 0.0)
            wv = wh2[:]
            z = bh2v[:][0]
            for m2 in range(16):
                z = z + hh[m2] * wv[m2]
            sig = 1.0 / (1.0 + jnp.exp(jnp.full((16,), -z, F32)))
            o0 = jnp.where(lanes == n, sig, o0)
            o1 = jnp.where(lanes + 16 == n, sig, o1)
            return (o0, o1)
        o0, o1 = lax.fori_loop(0, K, head_n, (zeros, zeros))
        outv[0:16] = o0
        outv[16:32] = o1
        pltpu.sync_copy(outv, out_hbm.at[g])

    def gloop(gs, c):
        g = wid + gs * NW

        @pl.when(g < B)
        def _():
            do_graph(g)
        return c
    lax.fori_loop(0, 2, gloop, 0)


def kernel(x, edge_index, edge_attr,
           W1a, b1a, W1b, b1b, W2a, b2a, W2b, b2b,
           Wh1, bh1, Wh2, bh2):
    xp = jnp.pad(x, ((0, 0), (0, 7))).reshape(B, K, 16)
    idx = []
    msk = []
    for p in range(K * 32):
        j, i = divmod(p, 32)
        if i < K and i != j:
            idx.append(i * (K - 1) + (j if j < i else j - 1))
            msk.append(True)
        else:
            idx.append(0)
            msk.append(False)
    for jp in range(5):
        for lane in range(16):
            s, d = divmod(lane, 4)
            j, i = 4 * jp + s, 16 + d
            if i != j:
                idx.append(i * (K - 1) + (j if j < i else j - 1))
                msk.append(True)
            else:
                idx.append(0)
                msk.append(False)
    ea = edge_attr.reshape(B, K * (K - 1))
    ap = jnp.where(jnp.asarray(msk)[None, :],
                   ea[:, jnp.asarray(idx, jnp.int32)], 0.0)
    w2bs = jnp.zeros((16, 16), F32).at[:, 1:9].set(W2b)
    b2bs = jnp.zeros((16,), F32).at[1:9].set(b2b)
    bh2v = jnp.full((16,), bh2[0], F32)

    mesh = plsc.VectorSubcoreMesh(core_axis_name="c", subcore_axis_name="s")
    run = functools.partial(
        pl.kernel, mesh=mesh,
        out_type=jax.ShapeDtypeStruct((B, 32), F32),
        scratch_types=[
            pltpu.VMEM((K, 16), F32),
            pltpu.VMEM((720,), F32),
            pltpu.VMEM((K, 32), F32),
            pltpu.VMEM((512,), F32),
            pltpu.VMEM((512,), F32),
            pltpu.VMEM((32 * 32,), F32),
            pltpu.VMEM((10, 32), F32),
            pltpu.VMEM((32,), F32),
            pltpu.VMEM((32, 32), F32),
            pltpu.VMEM((32,), F32),
            pltpu.VMEM((41, 16), F32),
            pltpu.VMEM((16,), F32),
            pltpu.VMEM((16, 16), F32),
            pltpu.VMEM((16,), F32),
            pltpu.VMEM((8, 16), F32),
            pltpu.VMEM((16,), F32),
            pltpu.VMEM((16,), F32),
            pltpu.VMEM((16,), F32),
            pltpu.VMEM((32,), F32),
        ],
    )(_body)
    out = run(xp, ap, W1a, b1a, W1b.T, b1b, W2a, b2a, w2bs, b2bs,
              Wh1, bh1, Wh2.reshape(16), bh2v)
    return out[:, :K].reshape(N, 1)
